# baseline (speedup 1.0000x reference)
import functools

import jax
import jax.numpy as jnp
from jax import lax
from jax.experimental import pallas as pl
from jax.experimental.pallas import tpu as pltpu
from jax.experimental.pallas import tpu_sc as plsc

_HW = 1024
_NCLS = 150
_NPAD = 152
_C = 96
_TAU = 0.3
_NCORES = 2
_BB = 8


def _scan_body(prob_ref, idx_ref, mask_ref):
    for i in range(_BB):
        p = prob_ref[i]
        m = jnp.max(p, axis=0, keepdims=True)
        ones_row = jnp.ones((1, _HW), jnp.float32)
        s = jax.lax.dot_general(ones_row, p, (((1,), (0,)), ((), ())),
                                preferred_element_type=jnp.float32)
        hw_iota = jax.lax.broadcasted_iota(jnp.int32, p.shape, 0)
        idx = jnp.min(jnp.where(p == m, hw_iota, _HW), axis=0, keepdims=True)
        rep = (s * (1.0 / _HW)) > _TAU
        pad_i = jnp.zeros((1, _NPAD - _NCLS), jnp.int32)
        pad_f = jnp.zeros((1, _NPAD - _NCLS), jnp.float32)
        idx_ref[i] = jnp.concatenate([idx, pad_i], axis=1)
        mask_ref[i] = jnp.concatenate([rep.astype(jnp.float32), pad_f],
                                      axis=1)


def _sc_body(emb_hbm, idx_hbm, mask_hbm, out_hbm, ebuf, outb, idxv, maskv):
    wid = lax.axis_index("s") * _NCORES + lax.axis_index("c")

    for b in (2 * wid, 2 * wid + 1):
        pltpu.sync_copy(idx_hbm.at[pl.ds(b * _NPAD, _NPAD)],
                        idxv.at[pl.ds(0, _NPAD)])
        pltpu.sync_copy(mask_hbm.at[pl.ds(b * _NPAD, _NPAD)],
                        maskv.at[pl.ds(0, _NPAD)])

        for s in range(2):
            pltpu.sync_copy(emb_hbm.at[b, pl.ds(s * 512, 512), :], ebuf)

            def cls_step(n, carry, s=s):
                idxn = idxv[pl.ds(n, 16)][0]
                maskn = maskv[pl.ds(n, 16)][0]
                off = jnp.clip(idxn - s * 512, 0, 511)
                inr = idxn >= 512
                for k in range(_C // 16):
                    sl = pl.ds(k * 16, 16)
                    v = ebuf[off, sl] * maskn
                    if s == 0:
                        outb[n, sl] = v
                    else:
                        outb[n, sl] = jnp.where(inr, v, outb[n, sl])
                return carry

            lax.fori_loop(0, _NCLS, cls_step, 0)

        pltpu.sync_copy(outb.at[pl.ds(0, _NCLS)], out_hbm.at[b])


def kernel(emb, prob_map):
    B = emb.shape[0]
    emb_r = emb.reshape(B, _HW, _C)
    prob_r = prob_map.reshape(B, _HW, _NCLS)

    idx, mask = pl.pallas_call(
        _scan_body,
        grid=(B // _BB,),
        in_specs=[pl.BlockSpec((_BB, _HW, _NCLS), lambda b: (b, 0, 0))],
        out_specs=[
            pl.BlockSpec((_BB, 1, _NPAD), lambda b: (b, 0, 0)),
            pl.BlockSpec((_BB, 1, _NPAD), lambda b: (b, 0, 0)),
        ],
        out_shape=[
            jax.ShapeDtypeStruct((B, 1, _NPAD), jnp.int32),
            jax.ShapeDtypeStruct((B, 1, _NPAD), jnp.float32),
        ],
    )(prob_r)

    mesh = plsc.VectorSubcoreMesh(core_axis_name="c", subcore_axis_name="s")
    sc_fn = functools.partial(
        pl.kernel,
        mesh=mesh,
        out_type=jax.ShapeDtypeStruct((B, _NCLS, _C), jnp.float32),
        scratch_types=[
            pltpu.VMEM((512, _C), jnp.float32),
            pltpu.VMEM((152, _C), jnp.float32),
            pltpu.VMEM((176,), jnp.int32),
            pltpu.VMEM((176,), jnp.float32),
        ],
    )(_sc_body)
    return sc_fn(emb_r, idx.reshape(B * _NPAD), mask.reshape(B * _NPAD))


# speedup vs baseline: 2.1911x; 2.1911x over previous
import jax
import jax.numpy as jnp
from jax.experimental import pallas as pl

_H, _W, _C = 32, 32, 96
_HW = _H * _W
_NCLS = 150
_TAU = 0.3


def _body(prob_ref, emb_ref, out_ref):
  for i in range(8):
    p = prob_ref[i]
    e = emb_ref[i]
    m = jnp.max(p, axis=0, keepdims=True)
    ones_row = jnp.ones((1, _HW), jnp.float32)
    s = jax.lax.dot_general(ones_row, p, (((1,), (0,)), ((), ())),
                            preferred_element_type=jnp.float32)
    hw_iota = jax.lax.broadcasted_iota(jnp.int32, p.shape, 0)
    idx = jnp.min(jnp.where(p == m, hw_iota, _HW), axis=0, keepdims=True)
    rep = (s * (1.0 / _HW)) > _TAU
    onehot = ((hw_iota == idx) & rep).astype(jnp.float32)
    out_ref[i] = jax.lax.dot_general(
        onehot, e, (((0,), (0,)), ((), ())),
        preferred_element_type=jnp.float32,
    )


def kernel(emb, prob_map):
    B = emb.shape[0]
    emb_flat = emb.reshape(B, _HW, _C)
    prob_flat = prob_map.reshape(B, _HW, _NCLS)
    out = pl.pallas_call(
        _body,
        grid=(B // 8,),
        in_specs=[
            pl.BlockSpec((8, _HW, _NCLS), lambda b: (b, 0, 0)),
            pl.BlockSpec((8, _HW, _C), lambda b: (b, 0, 0)),
        ],
        out_specs=pl.BlockSpec((8, _NCLS, _C), lambda b: (b, 0, 0)),
        out_shape=jax.ShapeDtypeStruct((B, _NCLS, _C), jnp.float32),
    )(prob_flat, emb_flat)
    return out
